# baseline (speedup 1.0000x reference)
import functools

import jax
import jax.numpy as jnp
from jax import lax
from jax.experimental import pallas as pl
from jax.experimental.pallas import tpu as pltpu
from jax.experimental.pallas import tpu_sc as plsc

NUM_CODES = 512
CODE_DIM = 32
BETA = 0.25

_NC = 2
_NS = 16
_NW = _NC * _NS
_LANES = 16

_M = 4096


def _vq_step(x_ref, e_ref, et_ref, zq_ref, idx_ref, loss_ref):
    i = pl.program_id(0)
    nb = x_ref.shape[1]
    m = nb * x_ref.shape[2]
    xb = x_ref[...].reshape(CODE_DIM, m)
    emb = e_ref[...]
    embt = et_ref[...]

    dot2 = jnp.dot(emb + emb, xb, preferred_element_type=jnp.float32)
    z2 = jnp.sum(xb * xb, axis=0, keepdims=True)
    e2 = jnp.sum(emb * emb, axis=1, keepdims=True)
    dist = (z2 + e2) - dot2

    idx = jnp.argmin(dist, axis=0)
    idx_ref[0, 0, :] = idx

    iota_col = jax.lax.broadcasted_iota(jnp.int32, (NUM_CODES, 1), 0)
    onehot = (iota_col == idx[None, :]).astype(jnp.float32)
    zq = jnp.dot(embt, onehot, preferred_element_type=jnp.float32)
    zq_ref[...] = zq.reshape(zq_ref.shape)

    part_loss = jnp.sum((xb - zq) ** 2).reshape(1, 1)

    @pl.when(i == 0)
    def _init():
        loss_ref[...] = part_loss

    @pl.when(i > 0)
    def _acc():
        loss_ref[...] += part_loss


def _vq_body_alias(x_ref, e_ref, et_ref, zq_in_ref, zq_ref, idx_ref,
                   loss_ref):
    del zq_in_ref
    _vq_step(x_ref, e_ref, et_ref, zq_ref, idx_ref, loss_ref)


def _hist_body(idx_hbm, out_hbm, idx_v, bins_v, out_v, chunk):
    wid = lax.axis_index("s") * _NC + lax.axis_index("c")
    base = wid * chunk
    pltpu.sync_copy(idx_hbm.at[pl.ds(base, chunk)], idx_v)

    zeros = jnp.zeros((_LANES,), jnp.float32)
    for i in range((_LANES * NUM_CODES) // _LANES):
        bins_v[pl.ds(i * _LANES, _LANES)] = zeros

    lane_off = lax.iota(jnp.int32, _LANES) * NUM_CODES
    ones = jnp.ones((_LANES,), jnp.float32)
    for i in range(chunk // _LANES):
        v = idx_v[pl.ds(i * _LANES, _LANES)]
        plsc.addupdate_scatter(bins_v, [lane_off + v], ones)

    for cg in range(NUM_CODES // _LANES):
        acc = bins_v[pl.ds(cg * _LANES, _LANES)]
        for r in range(1, _LANES):
            acc = acc + bins_v[pl.ds(r * NUM_CODES + cg * _LANES, _LANES)]
        out_v[pl.ds(cg * _LANES, _LANES)] = acc
    pltpu.sync_copy(out_v, out_hbm.at[wid])


def _epilogue_body(pa_ref, pb_ref, la_ref, lb_ref, avg_ref, perp_ref,
                   usage_ref, vql_ref, *, n_points, n_elems):
    counts = (jnp.sum(pa_ref[...], axis=0, keepdims=True)
              + jnp.sum(pb_ref[...], axis=0, keepdims=True))
    avg = counts * (1.0 / n_points)
    avg_ref[...] = avg
    ent = -jnp.sum(avg * jnp.log(avg + 1e-12))
    perp_ref[...] = jnp.exp(ent).reshape(1, 1)
    usage_ref[...] = jnp.mean((avg > 0).astype(jnp.float32)).reshape(1, 1)
    vql_ref[...] = BETA * (la_ref[...] + lb_ref[...]) / n_elems


def _make_hist(chunk):
    return pl.kernel(
        functools.partial(_hist_body, chunk=chunk),
        out_type=jax.ShapeDtypeStruct((_NW, NUM_CODES), jnp.float32),
        mesh=plsc.VectorSubcoreMesh(core_axis_name="c", subcore_axis_name="s",
                                    num_cores=_NC, num_subcores=_NS),
        scratch_types=[
            pltpu.VMEM((chunk,), jnp.int32),
            pltpu.VMEM((_LANES * NUM_CODES,), jnp.float32),
            pltpu.VMEM((NUM_CODES,), jnp.float32),
        ],
        compiler_params=pltpu.CompilerParams(needs_layout_passes=False),
    )


@jax.jit
def kernel(z_e, embedding):
    B, D, H, T = z_e.shape
    K = embedding.shape[0]
    N = H * T
    n_points = B * N
    x3 = z_e.reshape(B, D, N).transpose(1, 2, 0)
    embt = embedding.T

    nb = _M // B
    n_steps = n_points // _M
    half = n_steps // 2

    common_in = [
        pl.BlockSpec((K, D), lambda i: (0, 0)),
        pl.BlockSpec((D, K), lambda i: (0, 0)),
    ]
    out_shapes = (
        jax.ShapeDtypeStruct((D, N, B), jnp.float32),
        jax.ShapeDtypeStruct((half, 1, _M), jnp.int32),
        jax.ShapeDtypeStruct((1, 1), jnp.float32),
    )
    idx_loss_specs = (
        pl.BlockSpec((1, 1, _M), lambda i: (i, 0, 0)),
        pl.BlockSpec((1, 1), lambda i: (0, 0)),
    )

    zq_a, idx_a, loss_a = pl.pallas_call(
        _vq_step,
        grid=(half,),
        in_specs=[pl.BlockSpec((D, nb, B), lambda i: (0, i, 0))] + common_in,
        out_specs=(pl.BlockSpec((D, nb, B), lambda i: (0, i, 0)),)
        + idx_loss_specs,
        out_shape=out_shapes,
    )(x3, embedding, embt)

    zq3, idx_b, loss_b = pl.pallas_call(
        _vq_body_alias,
        grid=(half,),
        in_specs=[pl.BlockSpec((D, nb, B), lambda i: (0, i + half, 0))]
        + common_in
        + [pl.BlockSpec((D, nb, B), lambda i: (0, 0, 0))],
        out_specs=(pl.BlockSpec((D, nb, B), lambda i: (0, i + half, 0)),)
        + idx_loss_specs,
        out_shape=out_shapes,
        input_output_aliases={3: 0},
    )(x3, embedding, embt, zq_a)

    half_pts = n_points // 2
    chunk = half_pts // _NW
    hist = _make_hist(chunk)
    part_a = hist(idx_a.reshape(half_pts))
    part_b = hist(idx_b.reshape(half_pts))

    avg_probs, perp, usage, vq_loss = pl.pallas_call(
        functools.partial(_epilogue_body, n_points=n_points,
                          n_elems=n_points * D),
        in_specs=[
            pl.BlockSpec((_NW, K), lambda: (0, 0)),
            pl.BlockSpec((_NW, K), lambda: (0, 0)),
            pl.BlockSpec((1, 1), lambda: (0, 0)),
            pl.BlockSpec((1, 1), lambda: (0, 0)),
        ],
        out_specs=(
            pl.BlockSpec((1, K), lambda: (0, 0)),
            pl.BlockSpec((1, 1), lambda: (0, 0)),
            pl.BlockSpec((1, 1), lambda: (0, 0)),
            pl.BlockSpec((1, 1), lambda: (0, 0)),
        ),
        out_shape=(
            jax.ShapeDtypeStruct((1, K), jnp.float32),
            jax.ShapeDtypeStruct((1, 1), jnp.float32),
            jax.ShapeDtypeStruct((1, 1), jnp.float32),
            jax.ShapeDtypeStruct((1, 1), jnp.float32),
        ),
    )(part_a, part_b, loss_a, loss_b)

    idx_flat = jnp.concatenate([idx_a.reshape(half_pts),
                                idx_b.reshape(half_pts)])
    z_q_st = zq3.transpose(2, 0, 1).reshape(B, D, H, T)
    indices = idx_flat.reshape(N, B).T.reshape(B, H, T)
    return (z_q_st, vq_loss.reshape(()), indices, perp.reshape(()),
            usage.reshape(()), avg_probs.reshape(K))


# speedup vs baseline: 1.0485x; 1.0485x over previous
import functools

import jax
import jax.numpy as jnp
from jax import lax
from jax.experimental import pallas as pl
from jax.experimental.pallas import tpu as pltpu
from jax.experimental.pallas import tpu_sc as plsc

NUM_CODES = 512
CODE_DIM = 32
BETA = 0.25

_NC = 2
_NS = 16
_NW = _NC * _NS
_LANES = 16

_M = 8192


def _vq_body(x_ref, e_ref, et_ref, zq_ref, idx_ref, loss_ref):
    i = pl.program_id(0)
    nb = x_ref.shape[1]
    m = nb * x_ref.shape[2]
    xb = x_ref[...].reshape(CODE_DIM, m)
    emb = e_ref[...]
    embt = et_ref[...]

    dot2 = jnp.dot(emb + emb, xb, preferred_element_type=jnp.float32)
    z2 = jnp.sum(xb * xb, axis=0, keepdims=True)
    e2 = jnp.sum(emb * emb, axis=1, keepdims=True)
    dist = (z2 + e2) - dot2

    idx = jnp.argmin(dist, axis=0)
    idx_ref[0, 0, :] = idx

    iota_col = jax.lax.broadcasted_iota(jnp.int32, (NUM_CODES, 1), 0)
    onehot = (iota_col == idx[None, :]).astype(jnp.float32)
    zq = jnp.dot(embt, onehot, preferred_element_type=jnp.float32)
    zq_ref[...] = zq.reshape(zq_ref.shape)

    part_loss = jnp.sum((xb - zq) ** 2).reshape(1, 1)

    @pl.when(i == 0)
    def _init():
        loss_ref[...] = part_loss

    @pl.when(i > 0)
    def _acc():
        loss_ref[...] += part_loss


def _hist_body(idx_hbm, out_hbm, idx_v, bins_v, out_v, chunk):
    wid = lax.axis_index("s") * _NC + lax.axis_index("c")
    base = wid * chunk
    pltpu.sync_copy(idx_hbm.at[pl.ds(base, chunk)], idx_v)

    zeros = jnp.zeros((_LANES,), jnp.float32)
    for i in range((_LANES * NUM_CODES) // _LANES):
        bins_v[pl.ds(i * _LANES, _LANES)] = zeros

    lane_off = lax.iota(jnp.int32, _LANES) * NUM_CODES
    ones = jnp.ones((_LANES,), jnp.float32)
    for i in range(chunk // _LANES):
        v = idx_v[pl.ds(i * _LANES, _LANES)]
        plsc.addupdate_scatter(bins_v, [lane_off + v], ones)

    for cg in range(NUM_CODES // _LANES):
        acc = bins_v[pl.ds(cg * _LANES, _LANES)]
        for r in range(1, _LANES):
            acc = acc + bins_v[pl.ds(r * NUM_CODES + cg * _LANES, _LANES)]
        out_v[pl.ds(cg * _LANES, _LANES)] = acc
    pltpu.sync_copy(out_v, out_hbm.at[wid])


def _epilogue_body(part_ref, loss_ref, avg_ref, perp_ref, usage_ref,
                   vql_ref, *, n_points, n_elems):
    counts = jnp.sum(part_ref[...], axis=0, keepdims=True)
    avg = counts * (1.0 / n_points)
    avg_ref[...] = avg
    ent = -jnp.sum(avg * jnp.log(avg + 1e-12))
    perp_ref[...] = jnp.exp(ent).reshape(1, 1)
    usage_ref[...] = jnp.mean((avg > 0).astype(jnp.float32)).reshape(1, 1)
    vql_ref[...] = BETA * loss_ref[...] / n_elems


@jax.jit
def kernel(z_e, embedding):
    B, D, H, T = z_e.shape
    K = embedding.shape[0]
    N = H * T
    n_points = B * N
    x3 = z_e.reshape(B, D, N).transpose(1, 2, 0)
    embt = embedding.T

    nb = _M // B
    n_steps = n_points // _M
    zq3, idx3, loss_sum = pl.pallas_call(
        _vq_body,
        grid=(n_steps,),
        in_specs=[
            pl.BlockSpec((D, nb, B), lambda i: (0, i, 0)),
            pl.BlockSpec((K, D), lambda i: (0, 0)),
            pl.BlockSpec((D, K), lambda i: (0, 0)),
        ],
        out_specs=(
            pl.BlockSpec((D, nb, B), lambda i: (0, i, 0)),
            pl.BlockSpec((1, 1, _M), lambda i: (i, 0, 0)),
            pl.BlockSpec((1, 1), lambda i: (0, 0)),
        ),
        out_shape=(
            jax.ShapeDtypeStruct((D, N, B), jnp.float32),
            jax.ShapeDtypeStruct((n_steps, 1, _M), jnp.int32),
            jax.ShapeDtypeStruct((1, 1), jnp.float32),
        ),
        compiler_params=pltpu.CompilerParams(
            vmem_limit_bytes=120 * 1024 * 1024),
    )(x3, embedding, embt)

    idx_flat = idx3.reshape(n_points)

    chunk = n_points // _NW
    hist_kernel = pl.kernel(
        functools.partial(_hist_body, chunk=chunk),
        out_type=jax.ShapeDtypeStruct((_NW, K), jnp.float32),
        mesh=plsc.VectorSubcoreMesh(core_axis_name="c", subcore_axis_name="s",
                                    num_cores=_NC, num_subcores=_NS),
        scratch_types=[
            pltpu.VMEM((chunk,), jnp.int32),
            pltpu.VMEM((_LANES * K,), jnp.float32),
            pltpu.VMEM((K,), jnp.float32),
        ],
        compiler_params=pltpu.CompilerParams(needs_layout_passes=False),
    )
    hist_part = hist_kernel(idx_flat)

    avg_probs, perp, usage, vq_loss = pl.pallas_call(
        functools.partial(_epilogue_body, n_points=n_points,
                          n_elems=n_points * D),
        in_specs=[
            pl.BlockSpec((_NW, K), lambda: (0, 0)),
            pl.BlockSpec((1, 1), lambda: (0, 0)),
        ],
        out_specs=(
            pl.BlockSpec((1, K), lambda: (0, 0)),
            pl.BlockSpec((1, 1), lambda: (0, 0)),
            pl.BlockSpec((1, 1), lambda: (0, 0)),
            pl.BlockSpec((1, 1), lambda: (0, 0)),
        ),
        out_shape=(
            jax.ShapeDtypeStruct((1, K), jnp.float32),
            jax.ShapeDtypeStruct((1, 1), jnp.float32),
            jax.ShapeDtypeStruct((1, 1), jnp.float32),
            jax.ShapeDtypeStruct((1, 1), jnp.float32),
        ),
    )(hist_part, loss_sum)

    z_q_st = zq3.transpose(2, 0, 1).reshape(B, D, H, T)
    indices = idx_flat.reshape(N, B).T.reshape(B, H, T)
    return (z_q_st, vq_loss.reshape(()), indices, perp.reshape(()),
            usage.reshape(()), avg_probs.reshape(K))
